# baseline (device time: 23394 ns/iter reference)
import functools

import jax
import jax.numpy as jnp
from jax import lax
from jax.experimental import pallas as pl
from jax.experimental.pallas import tpu as pltpu

N_DEV = 16


def _mod2(c):
    return c - 2.0 * jnp.floor(c * 0.5)


def kernel(x):
    m, n = x.shape

    def body(x_ref, out_ref, comm_ref, send_buf, send_sems, recv_sems):
        my = lax.axis_index("i")

        xv = x_ref[:, :]
        la = jnp.log(jnp.abs(xv))
        neg = jnp.where(xv < 0.0, 1.0, 0.0)
        b = jnp.concatenate([la, neg], axis=1)
        r = lax.broadcasted_iota(jnp.int32, (m, m), 0)
        c = lax.broadcasted_iota(jnp.int32, (m, m), 1)
        tril = jnp.where(r >= c, 1.0, 0.0)
        s = jax.lax.dot(tril, b, precision=jax.lax.Precision.HIGHEST)

        send_buf[:, :] = s[m - 8 :, :]
        comm_ref[:, :, :] = jnp.zeros((N_DEV, 8, 2 * n), jnp.float32)

        barrier = pltpu.get_barrier_semaphore()
        for j in range(N_DEV):
            pl.semaphore_signal(
                barrier, inc=1,
                device_id=(j,), device_id_type=pl.DeviceIdType.MESH,
            )
        pl.semaphore_wait(barrier, N_DEV)

        for j in range(N_DEV):
            @pl.when(my < j)
            def _():
                rdma = pltpu.make_async_remote_copy(
                    src_ref=send_buf,
                    dst_ref=comm_ref.at[my],
                    send_sem=send_sems.at[j],
                    recv_sem=recv_sems.at[my],
                    device_id=(j,),
                    device_id_type=pl.DeviceIdType.MESH,
                )
                rdma.start()

        e_local = jnp.exp(s[:, :n])
        sign_local = 1.0 - 2.0 * _mod2(s[:, n:])

        for j in range(N_DEV):
            @pl.when(j < my)
            def _():
                recv = pltpu.make_async_remote_copy(
                    src_ref=send_buf,
                    dst_ref=comm_ref.at[j],
                    send_sem=send_sems.at[j],
                    recv_sem=recv_sems.at[j],
                    device_id=(j,),
                    device_id_type=pl.DeviceIdType.MESH,
                )
                recv.wait_recv()

        p = jnp.zeros((1, 2 * n), jnp.float32)
        for j in range(N_DEV):
            slot = comm_ref[j]
            p = p + slot[7:8, :]
        scale = jnp.exp(p[:, :n]) * (1.0 - 2.0 * _mod2(p[:, n:]))
        out_ref[:, :] = e_local * sign_local * scale

        for j in range(N_DEV):
            @pl.when(my < j)
            def _():
                rdma = pltpu.make_async_remote_copy(
                    src_ref=send_buf,
                    dst_ref=comm_ref.at[my],
                    send_sem=send_sems.at[j],
                    recv_sem=recv_sems.at[my],
                    device_id=(j,),
                    device_id_type=pl.DeviceIdType.MESH,
                )
                rdma.wait_send()

        @functools.partial(
            pl.run_scoped, sem2=pltpu.SemaphoreType.REGULAR
        )
        def _(sem2):
            for j in range(N_DEV):
                pl.semaphore_signal(
                    sem2, inc=1,
                    device_id=(j,), device_id_type=pl.DeviceIdType.MESH,
                )
            pl.semaphore_wait(sem2, N_DEV)

    return pl.pallas_call(
        body,
        out_shape=jax.ShapeDtypeStruct((m, n), jnp.float32),
        in_specs=[pl.BlockSpec(memory_space=pltpu.VMEM)],
        out_specs=pl.BlockSpec(memory_space=pltpu.VMEM),
        scratch_shapes=[
            pltpu.VMEM((N_DEV, 8, 2 * n), jnp.float32),
            pltpu.VMEM((8, 2 * n), jnp.float32),
            pltpu.SemaphoreType.DMA((N_DEV,)),
            pltpu.SemaphoreType.DMA((N_DEV,)),
        ],
        compiler_params=pltpu.CompilerParams(collective_id=0),
    )(x)


# device time: 20406 ns/iter; 1.1464x vs baseline; 1.1464x over previous
import functools

import jax
import jax.numpy as jnp
from jax import lax
from jax.experimental import pallas as pl
from jax.experimental.pallas import tpu as pltpu

N_DEV = 16


def _mod2(c):
    return c - 2.0 * jnp.floor(c * 0.5)


def kernel(x):
    m, n = x.shape

    def body(x_ref, out_ref, comm_ref, send_buf, send_sems, recv_sems):
        my = lax.axis_index("i")

        comm_ref[:, :, :] = jnp.zeros((N_DEV, 8, 2 * n), jnp.float32)

        barrier = pltpu.get_barrier_semaphore()
        for j in range(N_DEV):
            pl.semaphore_signal(
                barrier, inc=1,
                device_id=(j,), device_id_type=pl.DeviceIdType.MESH,
            )

        xv = x_ref[:, :]
        la = jnp.log(jnp.abs(xv))
        neg = jnp.where(xv < 0.0, 1.0, 0.0)
        tot = jnp.concatenate(
            [jnp.sum(la, axis=0, keepdims=True),
             jnp.sum(neg, axis=0, keepdims=True)],
            axis=1,
        )
        send_buf[:, :] = jnp.broadcast_to(tot, (8, 2 * n))

        pl.semaphore_wait(barrier, N_DEV)
        for j in range(N_DEV):
            @pl.when(my < j)
            def _():
                rdma = pltpu.make_async_remote_copy(
                    src_ref=send_buf,
                    dst_ref=comm_ref.at[my],
                    send_sem=send_sems.at[j],
                    recv_sem=recv_sems.at[my],
                    device_id=(j,),
                    device_id_type=pl.DeviceIdType.MESH,
                )
                rdma.start()

        b = jnp.concatenate([la, neg], axis=1)
        r = lax.broadcasted_iota(jnp.int32, (m, m), 0)
        c = lax.broadcasted_iota(jnp.int32, (m, m), 1)
        tril = jnp.where(r >= c, 1.0, 0.0)
        s = jax.lax.dot(tril, b)
        local = jnp.exp(s[:, :n]) * (1.0 - 2.0 * _mod2(s[:, n:]))

        for j in range(N_DEV):
            @pl.when(j < my)
            def _():
                recv = pltpu.make_async_remote_copy(
                    src_ref=send_buf,
                    dst_ref=comm_ref.at[j],
                    send_sem=send_sems.at[j],
                    recv_sem=recv_sems.at[j],
                    device_id=(j,),
                    device_id_type=pl.DeviceIdType.MESH,
                )
                recv.wait_recv()

        p = jnp.zeros((1, 2 * n), jnp.float32)
        for j in range(N_DEV):
            slot = comm_ref[j]
            p = p + slot[7:8, :]
        scale = jnp.exp(p[:, :n]) * (1.0 - 2.0 * _mod2(p[:, n:]))
        out_ref[:, :] = local * scale

        @functools.partial(
            pl.run_scoped, sem2=pltpu.SemaphoreType.REGULAR
        )
        def _(sem2):
            for j in range(N_DEV):
                pl.semaphore_signal(
                    sem2, inc=1,
                    device_id=(j,), device_id_type=pl.DeviceIdType.MESH,
                )
            for j in range(N_DEV):
                @pl.when(my < j)
                def _():
                    rdma = pltpu.make_async_remote_copy(
                        src_ref=send_buf,
                        dst_ref=comm_ref.at[my],
                        send_sem=send_sems.at[j],
                        recv_sem=recv_sems.at[my],
                        device_id=(j,),
                        device_id_type=pl.DeviceIdType.MESH,
                    )
                    rdma.wait_send()
            pl.semaphore_wait(sem2, N_DEV)

    return pl.pallas_call(
        body,
        out_shape=jax.ShapeDtypeStruct((m, n), jnp.float32),
        in_specs=[pl.BlockSpec(memory_space=pltpu.VMEM)],
        out_specs=pl.BlockSpec(memory_space=pltpu.VMEM),
        scratch_shapes=[
            pltpu.VMEM((N_DEV, 8, 2 * n), jnp.float32),
            pltpu.VMEM((8, 2 * n), jnp.float32),
            pltpu.SemaphoreType.DMA((N_DEV,)),
            pltpu.SemaphoreType.DMA((N_DEV,)),
        ],
        compiler_params=pltpu.CompilerParams(collective_id=0),
    )(x)


# device time: 19617 ns/iter; 1.1925x vs baseline; 1.0402x over previous
import functools
import os

import jax
import jax.numpy as jnp
from jax import lax
from jax.experimental import pallas as pl
from jax.experimental.pallas import tpu as pltpu

N_DEV = 16
_MODE = os.environ.get("KERNEL_MODE", "full")


def _mod2(c):
    return c - 2.0 * jnp.floor(c * 0.5)


def kernel(x):
    m, n = x.shape

    if _MODE in ("compute", "shift"):
        def probe(x_ref, out_ref):
            xv = x_ref[:, :]
            if _MODE == "compute":
                la = jnp.log(jnp.abs(xv))
                neg = jnp.where(xv < 0.0, 1.0, 0.0)
                b = jnp.concatenate([la, neg], axis=1)
                r = lax.broadcasted_iota(jnp.int32, (m, m), 0)
                c = lax.broadcasted_iota(jnp.int32, (m, m), 1)
                tril = jnp.where(r >= c, 1.0, 0.0)
                s = jax.lax.dot(tril, b)
                out_ref[:, :] = jnp.exp(s[:, :n]) * (1.0 - 2.0 * _mod2(s[:, n:]))
            else:
                acc = xv
                row = lax.broadcasted_iota(jnp.int32, (m, n), 0)
                shift = 1
                while shift < m:
                    rolled = pltpu.roll(acc, shift, 0)
                    acc = acc * jnp.where(row >= shift, rolled, 1.0)
                    shift *= 2
                out_ref[:, :] = acc

        return pl.pallas_call(
            probe,
            out_shape=jax.ShapeDtypeStruct((m, n), jnp.float32),
            in_specs=[pl.BlockSpec(memory_space=pltpu.VMEM)],
            out_specs=pl.BlockSpec(memory_space=pltpu.VMEM),
        )(x)

    def body(x_ref, out_ref, comm_ref, send_buf, send_sems, recv_sems):
        my = lax.axis_index("i")

        comm_ref[:, :, :] = jnp.zeros((N_DEV, 8, 2 * n), jnp.float32)

        barrier = pltpu.get_barrier_semaphore()
        for j in range(N_DEV):
            pl.semaphore_signal(
                barrier, inc=1,
                device_id=(j,), device_id_type=pl.DeviceIdType.MESH,
            )

        xv = x_ref[:, :]
        la = jnp.log(jnp.abs(xv))
        neg = jnp.where(xv < 0.0, 1.0, 0.0)
        tot = jnp.concatenate(
            [jnp.sum(la, axis=0, keepdims=True),
             jnp.sum(neg, axis=0, keepdims=True)],
            axis=1,
        )
        send_buf[:, :] = jnp.broadcast_to(tot, (8, 2 * n))

        pl.semaphore_wait(barrier, N_DEV)
        for j in range(N_DEV):
            @pl.when(my < j)
            def _():
                rdma = pltpu.make_async_remote_copy(
                    src_ref=send_buf,
                    dst_ref=comm_ref.at[my],
                    send_sem=send_sems.at[j],
                    recv_sem=recv_sems.at[my],
                    device_id=(j,),
                    device_id_type=pl.DeviceIdType.MESH,
                )
                rdma.start()

        if _MODE == "comm":
            local = xv
        else:
            b = jnp.concatenate([la, neg], axis=1)
            r = lax.broadcasted_iota(jnp.int32, (m, m), 0)
            c = lax.broadcasted_iota(jnp.int32, (m, m), 1)
            tril = jnp.where(r >= c, 1.0, 0.0)
            s = jax.lax.dot(tril, b)
            local = jnp.exp(s[:, :n]) * (1.0 - 2.0 * _mod2(s[:, n:]))

        for j in range(N_DEV):
            @pl.when(j < my)
            def _():
                recv = pltpu.make_async_remote_copy(
                    src_ref=send_buf,
                    dst_ref=comm_ref.at[j],
                    send_sem=send_sems.at[j],
                    recv_sem=recv_sems.at[j],
                    device_id=(j,),
                    device_id_type=pl.DeviceIdType.MESH,
                )
                recv.wait_recv()

        p = jnp.zeros((1, 2 * n), jnp.float32)
        for j in range(N_DEV):
            slot = comm_ref[j]
            p = p + slot[7:8, :]
        scale = jnp.exp(p[:, :n]) * (1.0 - 2.0 * _mod2(p[:, n:]))
        out_ref[:, :] = local * scale

        @functools.partial(
            pl.run_scoped, sem2=pltpu.SemaphoreType.REGULAR
        )
        def _(sem2):
            for j in range(N_DEV):
                pl.semaphore_signal(
                    sem2, inc=1,
                    device_id=(j,), device_id_type=pl.DeviceIdType.MESH,
                )
            for j in range(N_DEV):
                @pl.when(my < j)
                def _():
                    rdma = pltpu.make_async_remote_copy(
                        src_ref=send_buf,
                        dst_ref=comm_ref.at[my],
                        send_sem=send_sems.at[j],
                        recv_sem=recv_sems.at[my],
                        device_id=(j,),
                        device_id_type=pl.DeviceIdType.MESH,
                    )
                    rdma.wait_send()
            pl.semaphore_wait(sem2, N_DEV)

    return pl.pallas_call(
        body,
        out_shape=jax.ShapeDtypeStruct((m, n), jnp.float32),
        in_specs=[pl.BlockSpec(memory_space=pltpu.VMEM)],
        out_specs=pl.BlockSpec(memory_space=pltpu.VMEM),
        scratch_shapes=[
            pltpu.VMEM((N_DEV, 8, 2 * n), jnp.float32),
            pltpu.VMEM((8, 2 * n), jnp.float32),
            pltpu.SemaphoreType.DMA((N_DEV,)),
            pltpu.SemaphoreType.DMA((N_DEV,)),
        ],
        compiler_params=pltpu.CompilerParams(collective_id=0),
    )(x)


# device time: 9109 ns/iter; 2.5682x vs baseline; 2.1536x over previous
import functools
import os

import jax
import jax.numpy as jnp
from jax import lax
from jax.experimental import pallas as pl
from jax.experimental.pallas import tpu as pltpu

N_DEV = 16
_MODE = os.environ.get("KERNEL_MODE", "full")


def _mod2(c):
    return c - 2.0 * jnp.floor(c * 0.5)


def kernel(x):
    m, n = x.shape

    if _MODE == "barrier":
        def probe(x_ref, out_ref):
            barrier = pltpu.get_barrier_semaphore()
            for j in range(N_DEV):
                pl.semaphore_signal(
                    barrier, inc=1,
                    device_id=(j,), device_id_type=pl.DeviceIdType.MESH,
                )
            pl.semaphore_wait(barrier, N_DEV)
            out_ref[:, :] = x_ref[:, :]

            @functools.partial(pl.run_scoped, sem2=pltpu.SemaphoreType.REGULAR)
            def _(sem2):
                for j in range(N_DEV):
                    pl.semaphore_signal(
                        sem2, inc=1,
                        device_id=(j,), device_id_type=pl.DeviceIdType.MESH,
                    )
                pl.semaphore_wait(sem2, N_DEV)

        return pl.pallas_call(
            probe,
            out_shape=jax.ShapeDtypeStruct((m, n), jnp.float32),
            in_specs=[pl.BlockSpec(memory_space=pltpu.VMEM)],
            out_specs=pl.BlockSpec(memory_space=pltpu.VMEM),
            compiler_params=pltpu.CompilerParams(collective_id=0),
        )(x)

    if _MODE in ("compute", "shift"):
        def probe(x_ref, out_ref):
            xv = x_ref[:, :]
            if _MODE == "compute":
                la = jnp.log(jnp.abs(xv))
                neg = jnp.where(xv < 0.0, 1.0, 0.0)
                b = jnp.concatenate([la, neg], axis=1)
                r = lax.broadcasted_iota(jnp.int32, (m, m), 0)
                c = lax.broadcasted_iota(jnp.int32, (m, m), 1)
                tril = jnp.where(r >= c, 1.0, 0.0)
                s = jax.lax.dot(tril, b)
                out_ref[:, :] = jnp.exp(s[:, :n]) * (1.0 - 2.0 * _mod2(s[:, n:]))
            else:
                acc = xv
                row = lax.broadcasted_iota(jnp.int32, (m, n), 0)
                shift = 1
                while shift < m:
                    rolled = pltpu.roll(acc, shift, 0)
                    acc = acc * jnp.where(row >= shift, rolled, 1.0)
                    shift *= 2
                out_ref[:, :] = acc

        return pl.pallas_call(
            probe,
            out_shape=jax.ShapeDtypeStruct((m, n), jnp.float32),
            in_specs=[pl.BlockSpec(memory_space=pltpu.VMEM)],
            out_specs=pl.BlockSpec(memory_space=pltpu.VMEM),
        )(x)

    def body(x_ref, out_ref, comm_ref, send_buf, send_sems, recv_sems):
        my = lax.axis_index("i")

        comm_ref[:, :, :] = jnp.zeros((N_DEV, 8, 2 * n), jnp.float32)

        barrier = pltpu.get_barrier_semaphore()
        for j in range(N_DEV):
            pl.semaphore_signal(
                barrier, inc=1,
                device_id=(j,), device_id_type=pl.DeviceIdType.MESH,
            )

        xv = x_ref[:, :]
        la = jnp.log(jnp.abs(xv))
        neg = jnp.where(xv < 0.0, 1.0, 0.0)
        tot = jnp.concatenate(
            [jnp.sum(la, axis=0, keepdims=True),
             jnp.sum(neg, axis=0, keepdims=True)],
            axis=1,
        )
        send_buf[:, :] = jnp.broadcast_to(tot, (8, 2 * n))

        pl.semaphore_wait(barrier, N_DEV)
        for j in range(N_DEV):
            @pl.when(my < j)
            def _():
                rdma = pltpu.make_async_remote_copy(
                    src_ref=send_buf,
                    dst_ref=comm_ref.at[my],
                    send_sem=send_sems.at[j],
                    recv_sem=recv_sems.at[my],
                    device_id=(j,),
                    device_id_type=pl.DeviceIdType.MESH,
                )
                rdma.start()

        if _MODE == "comm":
            local = xv
        else:
            b = jnp.concatenate([la, neg], axis=1)
            r = lax.broadcasted_iota(jnp.int32, (m, m), 0)
            c = lax.broadcasted_iota(jnp.int32, (m, m), 1)
            tril = jnp.where(r >= c, 1.0, 0.0)
            s = jax.lax.dot(tril, b)
            local = jnp.exp(s[:, :n]) * (1.0 - 2.0 * _mod2(s[:, n:]))

        for j in range(N_DEV):
            @pl.when(j < my)
            def _():
                recv = pltpu.make_async_remote_copy(
                    src_ref=send_buf,
                    dst_ref=comm_ref.at[j],
                    send_sem=send_sems.at[j],
                    recv_sem=recv_sems.at[j],
                    device_id=(j,),
                    device_id_type=pl.DeviceIdType.MESH,
                )
                recv.wait_recv()

        p = jnp.zeros((1, 2 * n), jnp.float32)
        for j in range(N_DEV):
            slot = comm_ref[j]
            p = p + slot[7:8, :]
        scale = jnp.exp(p[:, :n]) * (1.0 - 2.0 * _mod2(p[:, n:]))
        out_ref[:, :] = local * scale

        @functools.partial(
            pl.run_scoped, sem2=pltpu.SemaphoreType.REGULAR
        )
        def _(sem2):
            for j in range(N_DEV):
                pl.semaphore_signal(
                    sem2, inc=1,
                    device_id=(j,), device_id_type=pl.DeviceIdType.MESH,
                )
            for j in range(N_DEV):
                @pl.when(my < j)
                def _():
                    rdma = pltpu.make_async_remote_copy(
                        src_ref=send_buf,
                        dst_ref=comm_ref.at[my],
                        send_sem=send_sems.at[j],
                        recv_sem=recv_sems.at[my],
                        device_id=(j,),
                        device_id_type=pl.DeviceIdType.MESH,
                    )
                    rdma.wait_send()
            pl.semaphore_wait(sem2, N_DEV)

    return pl.pallas_call(
        body,
        out_shape=jax.ShapeDtypeStruct((m, n), jnp.float32),
        in_specs=[pl.BlockSpec(memory_space=pltpu.VMEM)],
        out_specs=pl.BlockSpec(memory_space=pltpu.VMEM),
        scratch_shapes=[
            pltpu.VMEM((N_DEV, 8, 2 * n), jnp.float32),
            pltpu.VMEM((8, 2 * n), jnp.float32),
            pltpu.SemaphoreType.DMA((N_DEV,)),
            pltpu.SemaphoreType.DMA((N_DEV,)),
        ],
        compiler_params=pltpu.CompilerParams(collective_id=0),
    )(x)
